# Initial kernel scaffold; baseline (speedup 1.0000x reference)
import jax
import jax.numpy as jnp
from jax.experimental import pallas as pl


def kernel(feats, edge_index, node_graph_ids, W1, b1, Wres1, bres1, W2, b2, Wres2, bres2, w_atom, b_atom):
    raise NotImplementedError("write your pallas kernel here")


# speedup vs baseline: 4.3786x; 4.3786x over previous
import functools

import jax
import jax.numpy as jnp
from jax import lax
from jax.experimental import pallas as pl
from jax.experimental.pallas import tpu as pltpu
from jax.experimental.pallas import tpu_sc as plsc

_N = 10000
_E = 320000
_G = 64
_F = 128
_H = 256
_R = 1000
_NB = _N // _R
_NS = 16
_C = 80
_GR = 40
_NG = _N // _GR


def _sc_mesh():
    return plsc.VectorSubcoreMesh(core_axis_name="c", subcore_axis_name="s")


def _sc_scratch(feat_dim):
    return [
        pltpu.VMEM((_C,), jnp.int32),
        pltpu.VMEM((_C,), jnp.int32),
        pltpu.VMEM((_C, feat_dim), jnp.float32),
        pltpu.VMEM((_GR, feat_dim), jnp.float32),
        pltpu.VMEM_SHARED((_N, feat_dim), jnp.float32),
        pltpu.SemaphoreType.DMA,
    ]


def _group_range(s):
    return _NG * s // _NS, _NG * (s + 1) // _NS


def _zero_acc(zero_v, acc_sh, s, feat_dim):
    @pl.loop(0, _GR)
    def _(r):
        for j in range(feat_dim // 16):
            zero_v[r, pl.ds(j * 16, 16)] = jnp.zeros((16,), jnp.float32)

    g0, g1 = _group_range(s)

    @pl.loop(g0, g1)
    def _(g):
        pltpu.sync_copy(zero_v, acc_sh.at[pl.ds(g * _GR, _GR)])


def _writeout(acc_sh, out_hbm, s):
    g0, g1 = _group_range(s)

    @pl.loop(g0, g1)
    def _(g):
        pltpu.sync_copy(acc_sh.at[pl.ds(g * _GR, _GR)],
                        out_hbm.at[pl.ds(g * _GR, _GR)])


def _sc_agg_feats(feats, src, dst):
    e_core = _E // 2
    e_sub = e_core // _NS
    nchunks = e_sub // _C

    @functools.partial(
        pl.kernel,
        out_type=(jax.ShapeDtypeStruct((_N, _F), jnp.float32),
                  jax.ShapeDtypeStruct((_N, _F), jnp.float32)),
        mesh=_sc_mesh(),
        scratch_types=_sc_scratch(_F),
    )
    def k(feats_hbm, src_hbm, dst_hbm, p0_hbm, p1_hbm,
          src_v, dst_v, rows_v, zero_v, acc_sh, sem):
        c = lax.axis_index("c")
        s = lax.axis_index("s")
        _zero_acc(zero_v, acc_sh, s, _F)
        plsc.subcore_barrier()
        base0 = c * e_core + s * e_sub

        @pl.loop(0, nchunks)
        def _(i):
            base = base0 + i * _C
            pltpu.sync_copy(src_hbm.at[pl.ds(base, _C)], src_v)
            pltpu.sync_copy(dst_hbm.at[pl.ds(base, _C)], dst_v)
            pltpu.async_copy(feats_hbm.at[src_v], rows_v, sem).wait()
            pltpu.sync_copy(rows_v, acc_sh.at[dst_v], add=True)

        plsc.subcore_barrier()

        @pl.when(c == 0)
        def _():
            _writeout(acc_sh, p0_hbm, s)

        @pl.when(c == 1)
        def _():
            _writeout(acc_sh, p1_hbm, s)

    return k(feats, src, dst)


def _sc_agg_hidden(h_lo, h_hi, src, dst):
    e_sub = _E // _NS
    nchunks = e_sub // _C

    @functools.partial(
        pl.kernel,
        out_type=(jax.ShapeDtypeStruct((_N, _F), jnp.float32),
                  jax.ShapeDtypeStruct((_N, _F), jnp.float32)),
        mesh=_sc_mesh(),
        scratch_types=_sc_scratch(_F),
    )
    def k(hlo_hbm, hhi_hbm, src_hbm, dst_hbm, alo_hbm, ahi_hbm,
          src_v, dst_v, rows_v, zero_v, acc_sh, sem):
        c = lax.axis_index("c")
        s = lax.axis_index("s")
        _zero_acc(zero_v, acc_sh, s, _F)
        plsc.subcore_barrier()
        base0 = s * e_sub

        @pl.loop(0, nchunks)
        def _(i):
            base = base0 + i * _C
            pltpu.sync_copy(src_hbm.at[pl.ds(base, _C)], src_v)
            pltpu.sync_copy(dst_hbm.at[pl.ds(base, _C)], dst_v)

            @pl.when(c == 0)
            def _():
                pltpu.async_copy(hlo_hbm.at[src_v], rows_v, sem).wait()

            @pl.when(c == 1)
            def _():
                pltpu.async_copy(hhi_hbm.at[src_v], rows_v, sem).wait()

            pltpu.sync_copy(rows_v, acc_sh.at[dst_v], add=True)

        plsc.subcore_barrier()

        @pl.when(c == 0)
        def _():
            _writeout(acc_sh, alo_hbm, s)

        @pl.when(c == 1)
        def _():
            _writeout(acc_sh, ahi_hbm, s)

    return k(h_lo, h_hi, src, dst)


def _relu_mm_body(x_ref, w_ref, b_ref, o_ref):
    o_ref[...] = jnp.maximum(
        jnp.dot(x_ref[...], w_ref[...], preferred_element_type=jnp.float32,
                precision=lax.Precision.HIGHEST)
        + b_ref[...], 0.0)


def _tc_relu_mm(x, w, b):
    kdim = x.shape[1]
    return pl.pallas_call(
        _relu_mm_body,
        grid=(_NB,),
        in_specs=[
            pl.BlockSpec((_R, kdim), lambda i: (i, 0)),
            pl.BlockSpec((kdim, _H), lambda i: (0, 0)),
            pl.BlockSpec((1, _H), lambda i: (0, 0)),
        ],
        out_specs=pl.BlockSpec((_R, _H), lambda i: (i, 0)),
        out_shape=jax.ShapeDtypeStruct((_N, _H), jnp.float32),
    )(x, w, b)


def _res2_body(hlo_ref, hhi_ref, w_ref, b_ref, o_ref):
    w = w_ref[...]
    acc = (jnp.dot(hlo_ref[...], w[:_F, :], preferred_element_type=jnp.float32,
                precision=lax.Precision.HIGHEST)
           + jnp.dot(hhi_ref[...], w[_F:, :], preferred_element_type=jnp.float32,
                precision=lax.Precision.HIGHEST))
    o_ref[...] = jnp.maximum(acc + b_ref[...], 0.0)


def _tc_res2(h_lo, h_hi, w, b):
    return pl.pallas_call(
        _res2_body,
        grid=(_NB,),
        in_specs=[
            pl.BlockSpec((_R, _F), lambda i: (i, 0)),
            pl.BlockSpec((_R, _F), lambda i: (i, 0)),
            pl.BlockSpec((_H, _H), lambda i: (0, 0)),
            pl.BlockSpec((1, _H), lambda i: (0, 0)),
        ],
        out_specs=pl.BlockSpec((_R, _H), lambda i: (i, 0)),
        out_shape=jax.ShapeDtypeStruct((_N, _H), jnp.float32),
    )(h_lo, h_hi, w, b)


def _layer1_body(p0_ref, p1_ref, r1_ref, w_ref, b_ref, lo_ref, hi_ref):
    agg = p0_ref[...] + p1_ref[...]
    h = jnp.maximum(
        jnp.dot(agg, w_ref[...], preferred_element_type=jnp.float32,
                precision=lax.Precision.HIGHEST)
        + b_ref[...], 0.0) + r1_ref[...]
    lo_ref[...] = h[:, :_F]
    hi_ref[...] = h[:, _F:]


def _tc_layer1(p0, p1, r1, w, b):
    return pl.pallas_call(
        _layer1_body,
        grid=(_NB,),
        in_specs=[
            pl.BlockSpec((_R, _F), lambda i: (i, 0)),
            pl.BlockSpec((_R, _F), lambda i: (i, 0)),
            pl.BlockSpec((_R, _H), lambda i: (i, 0)),
            pl.BlockSpec((_F, _H), lambda i: (0, 0)),
            pl.BlockSpec((1, _H), lambda i: (0, 0)),
        ],
        out_specs=[
            pl.BlockSpec((_R, _F), lambda i: (i, 0)),
            pl.BlockSpec((_R, _F), lambda i: (i, 0)),
        ],
        out_shape=[
            jax.ShapeDtypeStruct((_N, _F), jnp.float32),
            jax.ShapeDtypeStruct((_N, _F), jnp.float32),
        ],
    )(p0, p1, r1, w, b)


def _final_body(alo_ref, ahi_ref, r2_ref, w2_ref, b2_ref, wa_ref, ba_ref,
                idrow_ref, idcol_ref, osum_ref, omax_ref):
    i = pl.program_id(0)
    w2 = w2_ref[...]
    h2 = jnp.maximum(
        jnp.dot(alo_ref[...], w2[:_F, :], preferred_element_type=jnp.float32,
                precision=lax.Precision.HIGHEST)
        + jnp.dot(ahi_ref[...], w2[_F:, :], preferred_element_type=jnp.float32,
                precision=lax.Precision.HIGHEST)
        + b2_ref[...], 0.0) + r2_ref[...]
    aw = jax.nn.sigmoid(
        jnp.sum(h2 * wa_ref[...], axis=1, keepdims=True) + ba_ref[...])

    ids_row = idrow_ref[0]
    idcol = idcol_ref[...]
    iota = lax.broadcasted_iota(jnp.int32, (_G, _R), 0)
    onehot = (iota == ids_row).astype(jnp.float32)
    wsum = jnp.dot(onehot, h2 * aw, preferred_element_type=jnp.float32,
                precision=lax.Precision.HIGHEST)

    x = h2
    sh = 1
    while sh < _R:
        xs = jnp.concatenate([jnp.zeros((sh, _H), jnp.float32), x[:_R - sh]],
                             axis=0)
        ics = jnp.concatenate(
            [jnp.full((sh, 1), -1, jnp.int32), idcol[:_R - sh]], axis=0)
        x = jnp.where(idcol == ics, jnp.maximum(x, xs), x)
        sh *= 2
    idn = jnp.concatenate([idcol[1:], jnp.full((1, 1), -1, jnp.int32)], axis=0)
    y = jnp.where(idcol != idn, x, 0.0)
    idn_row = jnp.concatenate(
        [ids_row[:, 1:], jnp.full((1, 1), -1, jnp.int32)], axis=1)
    flag_row = (ids_row != idn_row).astype(jnp.float32)
    onehot_f = onehot * flag_row
    cnt = jnp.sum(onehot_f, axis=1, keepdims=True)
    pmax = jnp.dot(onehot_f, y, preferred_element_type=jnp.float32,
                precision=lax.Precision.HIGHEST)
    pmax = jnp.where(cnt > 0.5, pmax, -jnp.inf)

    @pl.when(i == 0)
    def _():
        osum_ref[...] = jnp.zeros((_G, _H), jnp.float32)
        omax_ref[...] = jnp.full((_G, _H), -jnp.inf, jnp.float32)

    osum_ref[...] += wsum
    omax_ref[...] = jnp.maximum(omax_ref[...], pmax)


def _tc_final(a_lo, a_hi, r2, w2, b2, wa, ba, ids_row3, ids_col):
    return pl.pallas_call(
        _final_body,
        grid=(_NB,),
        in_specs=[
            pl.BlockSpec((_R, _F), lambda i: (i, 0)),
            pl.BlockSpec((_R, _F), lambda i: (i, 0)),
            pl.BlockSpec((_R, _H), lambda i: (i, 0)),
            pl.BlockSpec((_H, _H), lambda i: (0, 0)),
            pl.BlockSpec((1, _H), lambda i: (0, 0)),
            pl.BlockSpec((1, _H), lambda i: (0, 0)),
            pl.BlockSpec((1, 1), lambda i: (0, 0)),
            pl.BlockSpec((1, 1, _R), lambda i: (i, 0, 0)),
            pl.BlockSpec((_R, 1), lambda i: (i, 0)),
        ],
        out_specs=[
            pl.BlockSpec((_G, _H), lambda i: (0, 0)),
            pl.BlockSpec((_G, _H), lambda i: (0, 0)),
        ],
        out_shape=[
            jax.ShapeDtypeStruct((_G, _H), jnp.float32),
            jax.ShapeDtypeStruct((_G, _H), jnp.float32),
        ],
    )(a_lo, a_hi, r2, w2, b2, wa, ba, ids_row3, ids_col)


def kernel(feats, edge_index, node_graph_ids, W1, b1, Wres1, bres1,
           W2, b2, Wres2, bres2, w_atom, b_atom):
    src = edge_index[0]
    dst = edge_index[1]
    b1r = b1.reshape(1, _H)
    bres1r = bres1.reshape(1, _H)
    b2r = b2.reshape(1, _H)
    bres2r = bres2.reshape(1, _H)
    wa = w_atom.reshape(1, _H)
    bar = b_atom.reshape(1, 1)
    ids_row3 = node_graph_ids.reshape(_NB, 1, _R)
    ids_col = node_graph_ids.reshape(_N, 1)

    p0, p1 = _sc_agg_feats(feats, src, dst)
    r1 = _tc_relu_mm(feats, Wres1, bres1r)
    h_lo, h_hi = _tc_layer1(p0, p1, r1, W1, b1r)
    a_lo, a_hi = _sc_agg_hidden(h_lo, h_hi, src, dst)
    r2 = _tc_res2(h_lo, h_hi, Wres2, bres2r)
    osum, omax = _tc_final(a_lo, a_hi, r2, W2, b2r, wa, bar, ids_row3, ids_col)
    return jnp.concatenate([osum, omax], axis=1)


# speedup vs baseline: 8.4053x; 1.9196x over previous
import functools

import jax
import jax.numpy as jnp
from jax import lax
from jax.experimental import pallas as pl
from jax.experimental.pallas import tpu as pltpu
from jax.experimental.pallas import tpu_sc as plsc

_N = 10000
_E = 320000
_G = 64
_F = 128
_H = 256
_R = 1000
_NB = _N // _R
_NS = 16
_C = 80
_GR = 40
_NG = _N // _GR


def _sc_mesh():
    return plsc.VectorSubcoreMesh(core_axis_name="c", subcore_axis_name="s")


def _sc_scratch(feat_dim):
    return [
        pltpu.VMEM((_C,), jnp.int32),
        pltpu.VMEM((_C,), jnp.int32),
        pltpu.VMEM((_C,), jnp.int32),
        pltpu.VMEM((_C,), jnp.int32),
        pltpu.VMEM((_C, feat_dim), jnp.float32),
        pltpu.VMEM((_C, feat_dim), jnp.float32),
        pltpu.VMEM((_GR, feat_dim), jnp.float32),
        pltpu.VMEM_SHARED((_N, feat_dim), jnp.float32),
        pltpu.SemaphoreType.DMA,
        pltpu.SemaphoreType.DMA,
        pltpu.SemaphoreType.DMA,
        pltpu.SemaphoreType.DMA,
    ]


def _edge_pipeline(c, src_hbm, dst_hbm, base0, nchunks, tables, acc_sh,
                   srcv, dstv, rowsv, isems, gsems):
    def base(i):
        return base0 + i * _C

    def issue_idx(i, q):
        pltpu.make_async_copy(src_hbm.at[pl.ds(base(i), _C)],
                              srcv[q], isems[q]).start()
        pltpu.make_async_copy(dst_hbm.at[pl.ds(base(i), _C)],
                              dstv[q], isems[q]).start()

    def start_gather(i, q):
        pltpu.make_async_copy(src_hbm.at[pl.ds(base(i), _C)],
                              srcv[q], isems[q]).wait()
        pltpu.make_async_copy(dst_hbm.at[pl.ds(base(i), _C)],
                              dstv[q], isems[q]).wait()
        if len(tables) == 1:
            pltpu.make_async_copy(tables[0].at[srcv[q]],
                                  rowsv[q], gsems[q]).start()
        else:
            @pl.when(c == 0)
            def _():
                pltpu.make_async_copy(tables[0].at[srcv[q]],
                                      rowsv[q], gsems[q]).start()

            @pl.when(c == 1)
            def _():
                pltpu.make_async_copy(tables[1].at[srcv[q]],
                                      rowsv[q], gsems[q]).start()

    def scatter(i, q):
        if len(tables) == 1:
            pltpu.make_async_copy(tables[0].at[srcv[q]],
                                  rowsv[q], gsems[q]).wait()
        else:
            @pl.when(c == 0)
            def _():
                pltpu.make_async_copy(tables[0].at[srcv[q]],
                                      rowsv[q], gsems[q]).wait()

            @pl.when(c == 1)
            def _():
                pltpu.make_async_copy(tables[1].at[srcv[q]],
                                      rowsv[q], gsems[q]).wait()
        pltpu.sync_copy(rowsv[q], acc_sh.at[dstv[q]], add=True)

    issue_idx(0, 0)
    issue_idx(1, 1)
    start_gather(0, 0)
    kpairs = (nchunks - 2) // 2

    @pl.loop(0, kpairs)
    def _(p):
        i = 2 * p
        start_gather(i + 1, 1)
        scatter(i, 0)
        issue_idx(i + 2, 0)
        start_gather(i + 2, 0)
        scatter(i + 1, 1)
        issue_idx(i + 3, 1)

    for i in range(2 * kpairs, nchunks):
        if i + 1 < nchunks:
            start_gather(i + 1, (i + 1) % 2)
        scatter(i, i % 2)
        if i + 2 < nchunks:
            issue_idx(i + 2, (i + 2) % 2)


def _group_range(s):
    return _NG * s // _NS, _NG * (s + 1) // _NS


def _zero_acc(zero_v, acc_sh, s, feat_dim):
    @pl.loop(0, _GR)
    def _(r):
        for j in range(feat_dim // 16):
            zero_v[r, pl.ds(j * 16, 16)] = jnp.zeros((16,), jnp.float32)

    g0, g1 = _group_range(s)

    @pl.loop(g0, g1)
    def _(g):
        pltpu.sync_copy(zero_v, acc_sh.at[pl.ds(g * _GR, _GR)])


def _writeout(acc_sh, out_hbm, s):
    g0, g1 = _group_range(s)

    @pl.loop(g0, g1)
    def _(g):
        pltpu.sync_copy(acc_sh.at[pl.ds(g * _GR, _GR)],
                        out_hbm.at[pl.ds(g * _GR, _GR)])


def _sc_agg_feats(feats, src, dst):
    e_core = _E // 2
    e_sub = e_core // _NS
    nchunks = e_sub // _C

    @functools.partial(
        pl.kernel,
        out_type=(jax.ShapeDtypeStruct((_N, _F), jnp.float32),
                  jax.ShapeDtypeStruct((_N, _F), jnp.float32)),
        mesh=_sc_mesh(),
        scratch_types=_sc_scratch(_F),
    )
    def k(feats_hbm, src_hbm, dst_hbm, p0_hbm, p1_hbm,
          src0, src1, dst0, dst1, rows0, rows1, zero_v, acc_sh,
          isem0, isem1, gsem0, gsem1):
        c = lax.axis_index("c")
        s = lax.axis_index("s")
        _zero_acc(zero_v, acc_sh, s, _F)
        plsc.subcore_barrier()
        _edge_pipeline(c, src_hbm, dst_hbm, c * e_core + s * e_sub, nchunks,
                       (feats_hbm,), acc_sh, (src0, src1), (dst0, dst1),
                       (rows0, rows1), (isem0, isem1), (gsem0, gsem1))
        plsc.subcore_barrier()

        @pl.when(c == 0)
        def _():
            _writeout(acc_sh, p0_hbm, s)

        @pl.when(c == 1)
        def _():
            _writeout(acc_sh, p1_hbm, s)

    return k(feats, src, dst)


def _sc_agg_hidden(h_lo, h_hi, src, dst):
    e_sub = _E // _NS
    nchunks = e_sub // _C

    @functools.partial(
        pl.kernel,
        out_type=(jax.ShapeDtypeStruct((_N, _F), jnp.float32),
                  jax.ShapeDtypeStruct((_N, _F), jnp.float32)),
        mesh=_sc_mesh(),
        scratch_types=_sc_scratch(_F),
    )
    def k(hlo_hbm, hhi_hbm, src_hbm, dst_hbm, alo_hbm, ahi_hbm,
          src0, src1, dst0, dst1, rows0, rows1, zero_v, acc_sh,
          isem0, isem1, gsem0, gsem1):
        c = lax.axis_index("c")
        s = lax.axis_index("s")
        _zero_acc(zero_v, acc_sh, s, _F)
        plsc.subcore_barrier()
        _edge_pipeline(c, src_hbm, dst_hbm, s * e_sub, nchunks,
                       (hlo_hbm, hhi_hbm), acc_sh, (src0, src1), (dst0, dst1),
                       (rows0, rows1), (isem0, isem1), (gsem0, gsem1))
        plsc.subcore_barrier()

        @pl.when(c == 0)
        def _():
            _writeout(acc_sh, alo_hbm, s)

        @pl.when(c == 1)
        def _():
            _writeout(acc_sh, ahi_hbm, s)

    return k(h_lo, h_hi, src, dst)


def _relu_mm_body(x_ref, w_ref, b_ref, o_ref):
    o_ref[...] = jnp.maximum(
        jnp.dot(x_ref[...], w_ref[...], preferred_element_type=jnp.float32,
                precision=lax.Precision.HIGHEST)
        + b_ref[...], 0.0)


def _tc_relu_mm(x, w, b):
    kdim = x.shape[1]
    return pl.pallas_call(
        _relu_mm_body,
        grid=(_NB,),
        in_specs=[
            pl.BlockSpec((_R, kdim), lambda i: (i, 0)),
            pl.BlockSpec((kdim, _H), lambda i: (0, 0)),
            pl.BlockSpec((1, _H), lambda i: (0, 0)),
        ],
        out_specs=pl.BlockSpec((_R, _H), lambda i: (i, 0)),
        out_shape=jax.ShapeDtypeStruct((_N, _H), jnp.float32),
    )(x, w, b)


def _res2_body(hlo_ref, hhi_ref, w_ref, b_ref, o_ref):
    w = w_ref[...]
    acc = (jnp.dot(hlo_ref[...], w[:_F, :], preferred_element_type=jnp.float32,
                precision=lax.Precision.HIGHEST)
           + jnp.dot(hhi_ref[...], w[_F:, :], preferred_element_type=jnp.float32,
                precision=lax.Precision.HIGHEST))
    o_ref[...] = jnp.maximum(acc + b_ref[...], 0.0)


def _tc_res2(h_lo, h_hi, w, b):
    return pl.pallas_call(
        _res2_body,
        grid=(_NB,),
        in_specs=[
            pl.BlockSpec((_R, _F), lambda i: (i, 0)),
            pl.BlockSpec((_R, _F), lambda i: (i, 0)),
            pl.BlockSpec((_H, _H), lambda i: (0, 0)),
            pl.BlockSpec((1, _H), lambda i: (0, 0)),
        ],
        out_specs=pl.BlockSpec((_R, _H), lambda i: (i, 0)),
        out_shape=jax.ShapeDtypeStruct((_N, _H), jnp.float32),
    )(h_lo, h_hi, w, b)


def _layer1_body(p0_ref, p1_ref, r1_ref, w_ref, b_ref, lo_ref, hi_ref):
    agg = p0_ref[...] + p1_ref[...]
    h = jnp.maximum(
        jnp.dot(agg, w_ref[...], preferred_element_type=jnp.float32,
                precision=lax.Precision.HIGHEST)
        + b_ref[...], 0.0) + r1_ref[...]
    lo_ref[...] = h[:, :_F]
    hi_ref[...] = h[:, _F:]


def _tc_layer1(p0, p1, r1, w, b):
    return pl.pallas_call(
        _layer1_body,
        grid=(_NB,),
        in_specs=[
            pl.BlockSpec((_R, _F), lambda i: (i, 0)),
            pl.BlockSpec((_R, _F), lambda i: (i, 0)),
            pl.BlockSpec((_R, _H), lambda i: (i, 0)),
            pl.BlockSpec((_F, _H), lambda i: (0, 0)),
            pl.BlockSpec((1, _H), lambda i: (0, 0)),
        ],
        out_specs=[
            pl.BlockSpec((_R, _F), lambda i: (i, 0)),
            pl.BlockSpec((_R, _F), lambda i: (i, 0)),
        ],
        out_shape=[
            jax.ShapeDtypeStruct((_N, _F), jnp.float32),
            jax.ShapeDtypeStruct((_N, _F), jnp.float32),
        ],
    )(p0, p1, r1, w, b)


def _final_body(alo_ref, ahi_ref, r2_ref, w2_ref, b2_ref, wa_ref, ba_ref,
                idrow_ref, idcol_ref, osum_ref, omax_ref):
    i = pl.program_id(0)
    w2 = w2_ref[...]
    h2 = jnp.maximum(
        jnp.dot(alo_ref[...], w2[:_F, :], preferred_element_type=jnp.float32,
                precision=lax.Precision.HIGHEST)
        + jnp.dot(ahi_ref[...], w2[_F:, :], preferred_element_type=jnp.float32,
                precision=lax.Precision.HIGHEST)
        + b2_ref[...], 0.0) + r2_ref[...]
    aw = jax.nn.sigmoid(
        jnp.sum(h2 * wa_ref[...], axis=1, keepdims=True) + ba_ref[...])

    ids_row = idrow_ref[0]
    idcol = idcol_ref[...]
    iota = lax.broadcasted_iota(jnp.int32, (_G, _R), 0)
    onehot = (iota == ids_row).astype(jnp.float32)
    wsum = jnp.dot(onehot, h2 * aw, preferred_element_type=jnp.float32,
                precision=lax.Precision.HIGHEST)

    x = h2
    sh = 1
    while sh < _R:
        xs = jnp.concatenate([jnp.zeros((sh, _H), jnp.float32), x[:_R - sh]],
                             axis=0)
        ics = jnp.concatenate(
            [jnp.full((sh, 1), -1, jnp.int32), idcol[:_R - sh]], axis=0)
        x = jnp.where(idcol == ics, jnp.maximum(x, xs), x)
        sh *= 2
    idn = jnp.concatenate([idcol[1:], jnp.full((1, 1), -1, jnp.int32)], axis=0)
    y = jnp.where(idcol != idn, x, 0.0)
    idn_row = jnp.concatenate(
        [ids_row[:, 1:], jnp.full((1, 1), -1, jnp.int32)], axis=1)
    flag_row = (ids_row != idn_row).astype(jnp.float32)
    onehot_f = onehot * flag_row
    cnt = jnp.sum(onehot_f, axis=1, keepdims=True)
    pmax = jnp.dot(onehot_f, y, preferred_element_type=jnp.float32,
                precision=lax.Precision.HIGHEST)
    pmax = jnp.where(cnt > 0.5, pmax, -jnp.inf)

    @pl.when(i == 0)
    def _():
        osum_ref[...] = jnp.zeros((_G, _H), jnp.float32)
        omax_ref[...] = jnp.full((_G, _H), -jnp.inf, jnp.float32)

    osum_ref[...] += wsum
    omax_ref[...] = jnp.maximum(omax_ref[...], pmax)


def _tc_final(a_lo, a_hi, r2, w2, b2, wa, ba, ids_row3, ids_col):
    return pl.pallas_call(
        _final_body,
        grid=(_NB,),
        in_specs=[
            pl.BlockSpec((_R, _F), lambda i: (i, 0)),
            pl.BlockSpec((_R, _F), lambda i: (i, 0)),
            pl.BlockSpec((_R, _H), lambda i: (i, 0)),
            pl.BlockSpec((_H, _H), lambda i: (0, 0)),
            pl.BlockSpec((1, _H), lambda i: (0, 0)),
            pl.BlockSpec((1, _H), lambda i: (0, 0)),
            pl.BlockSpec((1, 1), lambda i: (0, 0)),
            pl.BlockSpec((1, 1, _R), lambda i: (i, 0, 0)),
            pl.BlockSpec((_R, 1), lambda i: (i, 0)),
        ],
        out_specs=[
            pl.BlockSpec((_G, _H), lambda i: (0, 0)),
            pl.BlockSpec((_G, _H), lambda i: (0, 0)),
        ],
        out_shape=[
            jax.ShapeDtypeStruct((_G, _H), jnp.float32),
            jax.ShapeDtypeStruct((_G, _H), jnp.float32),
        ],
    )(a_lo, a_hi, r2, w2, b2, wa, ba, ids_row3, ids_col)


def kernel(feats, edge_index, node_graph_ids, W1, b1, Wres1, bres1,
           W2, b2, Wres2, bres2, w_atom, b_atom):
    src = edge_index[0]
    dst = edge_index[1]
    b1r = b1.reshape(1, _H)
    bres1r = bres1.reshape(1, _H)
    b2r = b2.reshape(1, _H)
    bres2r = bres2.reshape(1, _H)
    wa = w_atom.reshape(1, _H)
    bar = b_atom.reshape(1, 1)
    ids_row3 = node_graph_ids.reshape(_NB, 1, _R)
    ids_col = node_graph_ids.reshape(_N, 1)

    p0, p1 = _sc_agg_feats(feats, src, dst)
    r1 = _tc_relu_mm(feats, Wres1, bres1r)
    h_lo, h_hi = _tc_layer1(p0, p1, r1, W1, b1r)
    a_lo, a_hi = _sc_agg_hidden(h_lo, h_hi, src, dst)
    r2 = _tc_res2(h_lo, h_hi, Wres2, bres2r)
    osum, omax = _tc_final(a_lo, a_hi, r2, W2, b2r, wa, bar, ids_row3, ids_col)
    return jnp.concatenate([osum, omax], axis=1)


# speedup vs baseline: 11.2297x; 1.3360x over previous
import functools

import jax
import jax.numpy as jnp
from jax import lax
from jax.experimental import pallas as pl
from jax.experimental.pallas import tpu as pltpu
from jax.experimental.pallas import tpu_sc as plsc

_N = 10000
_E = 320000
_G = 64
_F = 128
_H = 256
_R = 1000
_NB = _N // _R
_NS = 16
_C = 80
_GR = 40
_NG = _N // _GR


def _sc_mesh():
    return plsc.VectorSubcoreMesh(core_axis_name="c", subcore_axis_name="s")


_SR = 4
_SI = 8


def _sc_scratch(feat_dim):
    return [
        pltpu.VMEM((_SI, _C), jnp.int32),
        pltpu.VMEM((_SI, _C), jnp.int32),
        pltpu.VMEM((_SR, _C, feat_dim), jnp.float32),
        pltpu.VMEM((_GR, feat_dim), jnp.float32),
        pltpu.VMEM_SHARED((_N, feat_dim), jnp.float32),
    ] + [pltpu.SemaphoreType.DMA] * (_SI + _SR + _SR)


def _edge_pipeline(c, src_hbm, dst_hbm, base0, nchunks, tables, acc_sh,
                   srcv, dstv, rowsv, isems, gsems, ssems):
    n = nchunks

    def base(i):
        return base0 + i * _C

    def idx_copies(i, z):
        return (pltpu.make_async_copy(src_hbm.at[pl.ds(base(i), _C)],
                                      srcv.at[z], isems[z]),
                pltpu.make_async_copy(dst_hbm.at[pl.ds(base(i), _C)],
                                      dstv.at[z], isems[z]))

    def issue_idx(i, z):
        for cp in idx_copies(i, z):
            cp.start()

    def wait_idx(i, z):
        for cp in idx_copies(i, z):
            cp.wait()

    def gather_copy(t, z, q):
        return pltpu.make_async_copy(tables[t].at[srcv.at[z]],
                                     rowsv.at[q], gsems[q])

    def start_gather(z, q):
        if len(tables) == 1:
            gather_copy(0, z, q).start()
        else:
            @pl.when(c == 0)
            def _():
                gather_copy(0, z, q).start()

            @pl.when(c == 1)
            def _():
                gather_copy(1, z, q).start()

    def wait_gather(z, q):
        if len(tables) == 1:
            gather_copy(0, z, q).wait()
        else:
            @pl.when(c == 0)
            def _():
                gather_copy(0, z, q).wait()

            @pl.when(c == 1)
            def _():
                gather_copy(1, z, q).wait()

    def scatter_copy(q, z):
        return pltpu.make_async_copy(rowsv.at[q], acc_sh.at[dstv.at[z]],
                                     ssems[q])

    def body(i, u):
        if u >= 2 or not isinstance(i, int) or i >= 2:
            scatter_copy((u + 2) % _SR, (u + 6) % _SI).wait()
        if not isinstance(i, int) or i + 6 < n:
            issue_idx(i + 6, (u + 6) % _SI)
        if not isinstance(i, int) or i + 2 < n:
            wait_idx(i + 2, (u + 2) % _SI)
            start_gather((u + 2) % _SI, (u + 2) % _SR)
        wait_gather(u % _SI, u % _SR)
        scatter_copy(u % _SR, u % _SI).start(add=True)

    for j in range(6):
        issue_idx(j, j)
    for j in range(2):
        wait_idx(j, j)
        start_gather(j, j)
    for i in range(_SI):
        if i >= 2:
            scatter_copy((i + 2) % _SR, (i + 6) % _SI).wait()
        issue_idx(i + 6, (i + 6) % _SI)
        wait_idx(i + 2, (i + 2) % _SI)
        start_gather((i + 2) % _SI, (i + 2) % _SR)
        wait_gather(i % _SI, i % _SR)
        scatter_copy(i % _SR, i % _SI).start(add=True)
    nblocks = (n - 6 - _SI) // _SI

    @pl.loop(0, nblocks)
    def _(b):
        i0 = _SI + _SI * b
        for u in range(_SI):
            body(i0 + u, u)

    for i in range(_SI + _SI * nblocks, n):
        body(i, i % _SI)
    scatter_copy((n - 2) % _SR, (n - 2) % _SI).wait()
    scatter_copy((n - 1) % _SR, (n - 1) % _SI).wait()


def _group_range(s):
    return _NG * s // _NS, _NG * (s + 1) // _NS


def _zero_acc(zero_v, acc_sh, s, feat_dim):
    @pl.loop(0, _GR)
    def _(r):
        for j in range(feat_dim // 16):
            zero_v[r, pl.ds(j * 16, 16)] = jnp.zeros((16,), jnp.float32)

    g0, g1 = _group_range(s)

    @pl.loop(g0, g1)
    def _(g):
        pltpu.sync_copy(zero_v, acc_sh.at[pl.ds(g * _GR, _GR)])


def _writeout(acc_sh, out_hbm, s):
    g0, g1 = _group_range(s)

    @pl.loop(g0, g1)
    def _(g):
        pltpu.sync_copy(acc_sh.at[pl.ds(g * _GR, _GR)],
                        out_hbm.at[pl.ds(g * _GR, _GR)])


def _sc_agg_feats(feats, src, dst):
    e_core = _E // 2
    e_sub = e_core // _NS
    nchunks = e_sub // _C

    @functools.partial(
        pl.kernel,
        out_type=(jax.ShapeDtypeStruct((_N, _F), jnp.float32),
                  jax.ShapeDtypeStruct((_N, _F), jnp.float32)),
        mesh=_sc_mesh(),
        scratch_types=_sc_scratch(_F),
    )
    def k(feats_hbm, src_hbm, dst_hbm, p0_hbm, p1_hbm,
          srcv, dstv, rowsv, zero_v, acc_sh, *sems):
        c = lax.axis_index("c")
        s = lax.axis_index("s")
        _zero_acc(zero_v, acc_sh, s, _F)
        plsc.subcore_barrier()
        _edge_pipeline(c, src_hbm, dst_hbm, c * e_core + s * e_sub, nchunks,
                       (feats_hbm,), acc_sh, srcv, dstv, rowsv,
                       sems[:_SI], sems[_SI:_SI + _SR], sems[_SI + _SR:])
        plsc.subcore_barrier()

        @pl.when(c == 0)
        def _():
            _writeout(acc_sh, p0_hbm, s)

        @pl.when(c == 1)
        def _():
            _writeout(acc_sh, p1_hbm, s)

    return k(feats, src, dst)


def _sc_agg_hidden(h_lo, h_hi, src, dst):
    e_sub = _E // _NS
    nchunks = e_sub // _C

    @functools.partial(
        pl.kernel,
        out_type=(jax.ShapeDtypeStruct((_N, _F), jnp.float32),
                  jax.ShapeDtypeStruct((_N, _F), jnp.float32)),
        mesh=_sc_mesh(),
        scratch_types=_sc_scratch(_F),
    )
    def k(hlo_hbm, hhi_hbm, src_hbm, dst_hbm, alo_hbm, ahi_hbm,
          srcv, dstv, rowsv, zero_v, acc_sh, *sems):
        c = lax.axis_index("c")
        s = lax.axis_index("s")
        _zero_acc(zero_v, acc_sh, s, _F)
        plsc.subcore_barrier()
        _edge_pipeline(c, src_hbm, dst_hbm, s * e_sub, nchunks,
                       (hlo_hbm, hhi_hbm), acc_sh, srcv, dstv, rowsv,
                       sems[:_SI], sems[_SI:_SI + _SR], sems[_SI + _SR:])
        plsc.subcore_barrier()

        @pl.when(c == 0)
        def _():
            _writeout(acc_sh, alo_hbm, s)

        @pl.when(c == 1)
        def _():
            _writeout(acc_sh, ahi_hbm, s)

    return k(h_lo, h_hi, src, dst)


def _relu_mm_body(x_ref, w_ref, b_ref, o_ref):
    o_ref[...] = jnp.maximum(
        jnp.dot(x_ref[...], w_ref[...], preferred_element_type=jnp.float32,
                precision=lax.Precision.HIGHEST)
        + b_ref[...], 0.0)


def _tc_relu_mm(x, w, b):
    kdim = x.shape[1]
    return pl.pallas_call(
        _relu_mm_body,
        grid=(_NB,),
        in_specs=[
            pl.BlockSpec((_R, kdim), lambda i: (i, 0)),
            pl.BlockSpec((kdim, _H), lambda i: (0, 0)),
            pl.BlockSpec((1, _H), lambda i: (0, 0)),
        ],
        out_specs=pl.BlockSpec((_R, _H), lambda i: (i, 0)),
        out_shape=jax.ShapeDtypeStruct((_N, _H), jnp.float32),
    )(x, w, b)


def _res2_body(hlo_ref, hhi_ref, w_ref, b_ref, o_ref):
    w = w_ref[...]
    acc = (jnp.dot(hlo_ref[...], w[:_F, :], preferred_element_type=jnp.float32,
                precision=lax.Precision.HIGHEST)
           + jnp.dot(hhi_ref[...], w[_F:, :], preferred_element_type=jnp.float32,
                precision=lax.Precision.HIGHEST))
    o_ref[...] = jnp.maximum(acc + b_ref[...], 0.0)


def _tc_res2(h_lo, h_hi, w, b):
    return pl.pallas_call(
        _res2_body,
        grid=(_NB,),
        in_specs=[
            pl.BlockSpec((_R, _F), lambda i: (i, 0)),
            pl.BlockSpec((_R, _F), lambda i: (i, 0)),
            pl.BlockSpec((_H, _H), lambda i: (0, 0)),
            pl.BlockSpec((1, _H), lambda i: (0, 0)),
        ],
        out_specs=pl.BlockSpec((_R, _H), lambda i: (i, 0)),
        out_shape=jax.ShapeDtypeStruct((_N, _H), jnp.float32),
    )(h_lo, h_hi, w, b)


def _layer1_body(p0_ref, p1_ref, r1_ref, w_ref, b_ref, lo_ref, hi_ref):
    agg = p0_ref[...] + p1_ref[...]
    h = jnp.maximum(
        jnp.dot(agg, w_ref[...], preferred_element_type=jnp.float32,
                precision=lax.Precision.HIGHEST)
        + b_ref[...], 0.0) + r1_ref[...]
    lo_ref[...] = h[:, :_F]
    hi_ref[...] = h[:, _F:]


def _tc_layer1(p0, p1, r1, w, b):
    return pl.pallas_call(
        _layer1_body,
        grid=(_NB,),
        in_specs=[
            pl.BlockSpec((_R, _F), lambda i: (i, 0)),
            pl.BlockSpec((_R, _F), lambda i: (i, 0)),
            pl.BlockSpec((_R, _H), lambda i: (i, 0)),
            pl.BlockSpec((_F, _H), lambda i: (0, 0)),
            pl.BlockSpec((1, _H), lambda i: (0, 0)),
        ],
        out_specs=[
            pl.BlockSpec((_R, _F), lambda i: (i, 0)),
            pl.BlockSpec((_R, _F), lambda i: (i, 0)),
        ],
        out_shape=[
            jax.ShapeDtypeStruct((_N, _F), jnp.float32),
            jax.ShapeDtypeStruct((_N, _F), jnp.float32),
        ],
    )(p0, p1, r1, w, b)


def _final_body(alo_ref, ahi_ref, r2_ref, w2_ref, b2_ref, wa_ref, ba_ref,
                idrow_ref, idcol_ref, osum_ref, omax_ref):
    i = pl.program_id(0)
    w2 = w2_ref[...]
    h2 = jnp.maximum(
        jnp.dot(alo_ref[...], w2[:_F, :], preferred_element_type=jnp.float32,
                precision=lax.Precision.HIGHEST)
        + jnp.dot(ahi_ref[...], w2[_F:, :], preferred_element_type=jnp.float32,
                precision=lax.Precision.HIGHEST)
        + b2_ref[...], 0.0) + r2_ref[...]
    aw = jax.nn.sigmoid(
        jnp.sum(h2 * wa_ref[...], axis=1, keepdims=True) + ba_ref[...])

    ids_row = idrow_ref[0]
    idcol = idcol_ref[...]
    iota = lax.broadcasted_iota(jnp.int32, (_G, _R), 0)
    onehot = (iota == ids_row).astype(jnp.float32)
    wsum = jnp.dot(onehot, h2 * aw, preferred_element_type=jnp.float32,
                precision=lax.Precision.HIGHEST)

    x = h2
    sh = 1
    while sh < _R:
        xs = jnp.concatenate([jnp.zeros((sh, _H), jnp.float32), x[:_R - sh]],
                             axis=0)
        ics = jnp.concatenate(
            [jnp.full((sh, 1), -1, jnp.int32), idcol[:_R - sh]], axis=0)
        x = jnp.where(idcol == ics, jnp.maximum(x, xs), x)
        sh *= 2
    idn = jnp.concatenate([idcol[1:], jnp.full((1, 1), -1, jnp.int32)], axis=0)
    y = jnp.where(idcol != idn, x, 0.0)
    idn_row = jnp.concatenate(
        [ids_row[:, 1:], jnp.full((1, 1), -1, jnp.int32)], axis=1)
    flag_row = (ids_row != idn_row).astype(jnp.float32)
    onehot_f = onehot * flag_row
    cnt = jnp.sum(onehot_f, axis=1, keepdims=True)
    pmax = jnp.dot(onehot_f, y, preferred_element_type=jnp.float32,
                precision=lax.Precision.HIGHEST)
    pmax = jnp.where(cnt > 0.5, pmax, -jnp.inf)

    @pl.when(i == 0)
    def _():
        osum_ref[...] = jnp.zeros((_G, _H), jnp.float32)
        omax_ref[...] = jnp.full((_G, _H), -jnp.inf, jnp.float32)

    osum_ref[...] += wsum
    omax_ref[...] = jnp.maximum(omax_ref[...], pmax)


def _tc_final(a_lo, a_hi, r2, w2, b2, wa, ba, ids_row3, ids_col):
    return pl.pallas_call(
        _final_body,
        grid=(_NB,),
        in_specs=[
            pl.BlockSpec((_R, _F), lambda i: (i, 0)),
            pl.BlockSpec((_R, _F), lambda i: (i, 0)),
            pl.BlockSpec((_R, _H), lambda i: (i, 0)),
            pl.BlockSpec((_H, _H), lambda i: (0, 0)),
            pl.BlockSpec((1, _H), lambda i: (0, 0)),
            pl.BlockSpec((1, _H), lambda i: (0, 0)),
            pl.BlockSpec((1, 1), lambda i: (0, 0)),
            pl.BlockSpec((1, 1, _R), lambda i: (i, 0, 0)),
            pl.BlockSpec((_R, 1), lambda i: (i, 0)),
        ],
        out_specs=[
            pl.BlockSpec((_G, _H), lambda i: (0, 0)),
            pl.BlockSpec((_G, _H), lambda i: (0, 0)),
        ],
        out_shape=[
            jax.ShapeDtypeStruct((_G, _H), jnp.float32),
            jax.ShapeDtypeStruct((_G, _H), jnp.float32),
        ],
    )(a_lo, a_hi, r2, w2, b2, wa, ba, ids_row3, ids_col)


def kernel(feats, edge_index, node_graph_ids, W1, b1, Wres1, bres1,
           W2, b2, Wres2, bres2, w_atom, b_atom):
    src = edge_index[0]
    dst = edge_index[1]
    b1r = b1.reshape(1, _H)
    bres1r = bres1.reshape(1, _H)
    b2r = b2.reshape(1, _H)
    bres2r = bres2.reshape(1, _H)
    wa = w_atom.reshape(1, _H)
    bar = b_atom.reshape(1, 1)
    ids_row3 = node_graph_ids.reshape(_NB, 1, _R)
    ids_col = node_graph_ids.reshape(_N, 1)

    p0, p1 = _sc_agg_feats(feats, src, dst)
    r1 = _tc_relu_mm(feats, Wres1, bres1r)
    h_lo, h_hi = _tc_layer1(p0, p1, r1, W1, b1r)
    a_lo, a_hi = _sc_agg_hidden(h_lo, h_hi, src, dst)
    r2 = _tc_res2(h_lo, h_hi, Wres2, bres2r)
    osum, omax = _tc_final(a_lo, a_hi, r2, W2, b2r, wa, bar, ids_row3, ids_col)
    return jnp.concatenate([osum, omax], axis=1)


# speedup vs baseline: 11.6870x; 1.0407x over previous
import functools

import jax
import jax.numpy as jnp
from jax import lax
from jax.experimental import pallas as pl
from jax.experimental.pallas import tpu as pltpu
from jax.experimental.pallas import tpu_sc as plsc

_N = 10000
_E = 320000
_G = 64
_F = 128
_H = 256
_R = 1000
_NB = _N // _R
_NS = 16
_C = 80
_GR = 40
_NG = _N // _GR


def _sc_mesh():
    return plsc.VectorSubcoreMesh(core_axis_name="c", subcore_axis_name="s")


_SR = 4
_SI = 8
_GL = 2
_WL = _SR - _GL
_IL = _SI - _WL


def _sc_scratch(feat_dim):
    return [
        pltpu.VMEM((_SI, _C), jnp.int32),
        pltpu.VMEM((_SI, _C), jnp.int32),
        pltpu.VMEM((_SR, _C, feat_dim), jnp.float32),
        pltpu.VMEM((_GR, feat_dim), jnp.float32),
        pltpu.VMEM_SHARED((_N, feat_dim), jnp.float32),
    ] + [pltpu.SemaphoreType.DMA] * (_SI + _SR + _SR)


def _edge_pipeline(c, src_hbm, dst_hbm, base0, nchunks, tables, acc_sh,
                   srcv, dstv, rowsv, isems, gsems, ssems):
    n = nchunks

    def base(i):
        return base0 + i * _C

    def idx_copies(i, z):
        return (pltpu.make_async_copy(src_hbm.at[pl.ds(base(i), _C)],
                                      srcv.at[z], isems[z]),
                pltpu.make_async_copy(dst_hbm.at[pl.ds(base(i), _C)],
                                      dstv.at[z], isems[z]))

    def issue_idx(i, z):
        for cp in idx_copies(i, z):
            cp.start()

    def wait_idx(i, z):
        for cp in idx_copies(i, z):
            cp.wait()

    def gather_copy(t, z, q):
        return pltpu.make_async_copy(tables[t].at[srcv.at[z]],
                                     rowsv.at[q], gsems[q])

    def start_gather(z, q):
        if len(tables) == 1:
            gather_copy(0, z, q).start()
        else:
            @pl.when(c == 0)
            def _():
                gather_copy(0, z, q).start()

            @pl.when(c == 1)
            def _():
                gather_copy(1, z, q).start()

    def wait_gather(z, q):
        if len(tables) == 1:
            gather_copy(0, z, q).wait()
        else:
            @pl.when(c == 0)
            def _():
                gather_copy(0, z, q).wait()

            @pl.when(c == 1)
            def _():
                gather_copy(1, z, q).wait()

    def scatter_copy(q, z):
        return pltpu.make_async_copy(rowsv.at[q], acc_sh.at[dstv.at[z]],
                                     ssems[q])

    def body(i, u, static):
        if (not static) or i >= _WL:
            scatter_copy((u - _WL) % _SR, (u - _WL) % _SI).wait()
        if (not static) or i + _IL < n:
            issue_idx(i + _IL, (u + _IL) % _SI)
        if (not static) or i + _GL < n:
            wait_idx(i + _GL, (u + _GL) % _SI)
            start_gather((u + _GL) % _SI, (u + _GL) % _SR)
        wait_gather(u % _SI, u % _SR)
        scatter_copy(u % _SR, u % _SI).start(add=True)

    for j in range(_IL):
        issue_idx(j, j % _SI)
    for j in range(_GL):
        wait_idx(j, j % _SI)
        start_gather(j % _SI, j % _SR)
    for i in range(_SI):
        body(i, i % _SI, True)
    nblocks = (n - _SI - _IL) // _SI

    @pl.loop(0, nblocks)
    def _(b):
        i0 = _SI + _SI * b
        for u in range(_SI):
            body(i0 + u, u, False)

    for i in range(_SI + _SI * nblocks, n):
        body(i, i % _SI, True)
    for j in range(n - _WL, n):
        scatter_copy(j % _SR, j % _SI).wait()


def _group_range(s):
    return _NG * s // _NS, _NG * (s + 1) // _NS


def _zero_acc(zero_v, acc_sh, s, feat_dim):
    @pl.loop(0, _GR)
    def _(r):
        for j in range(feat_dim // 16):
            zero_v[r, pl.ds(j * 16, 16)] = jnp.zeros((16,), jnp.float32)

    g0, g1 = _group_range(s)

    @pl.loop(g0, g1)
    def _(g):
        pltpu.sync_copy(zero_v, acc_sh.at[pl.ds(g * _GR, _GR)])


def _writeout(acc_sh, out_hbm, s):
    g0, g1 = _group_range(s)

    @pl.loop(g0, g1)
    def _(g):
        pltpu.sync_copy(acc_sh.at[pl.ds(g * _GR, _GR)],
                        out_hbm.at[pl.ds(g * _GR, _GR)])


def _sc_agg_feats(feats, src, dst):
    e_core = _E // 2
    e_sub = e_core // _NS
    nchunks = e_sub // _C

    @functools.partial(
        pl.kernel,
        out_type=(jax.ShapeDtypeStruct((_N, _F), jnp.float32),
                  jax.ShapeDtypeStruct((_N, _F), jnp.float32)),
        mesh=_sc_mesh(),
        scratch_types=_sc_scratch(_F),
    )
    def k(feats_hbm, src_hbm, dst_hbm, p0_hbm, p1_hbm,
          srcv, dstv, rowsv, zero_v, acc_sh, *sems):
        c = lax.axis_index("c")
        s = lax.axis_index("s")
        _zero_acc(zero_v, acc_sh, s, _F)
        plsc.subcore_barrier()
        _edge_pipeline(c, src_hbm, dst_hbm, c * e_core + s * e_sub, nchunks,
                       (feats_hbm,), acc_sh, srcv, dstv, rowsv,
                       sems[:_SI], sems[_SI:_SI + _SR], sems[_SI + _SR:])
        plsc.subcore_barrier()

        @pl.when(c == 0)
        def _():
            _writeout(acc_sh, p0_hbm, s)

        @pl.when(c == 1)
        def _():
            _writeout(acc_sh, p1_hbm, s)

    return k(feats, src, dst)


def _sc_agg_hidden(h_lo, h_hi, src, dst):
    e_sub = _E // _NS
    nchunks = e_sub // _C

    @functools.partial(
        pl.kernel,
        out_type=(jax.ShapeDtypeStruct((_N, _F), jnp.float32),
                  jax.ShapeDtypeStruct((_N, _F), jnp.float32)),
        mesh=_sc_mesh(),
        scratch_types=_sc_scratch(_F),
    )
    def k(hlo_hbm, hhi_hbm, src_hbm, dst_hbm, alo_hbm, ahi_hbm,
          srcv, dstv, rowsv, zero_v, acc_sh, *sems):
        c = lax.axis_index("c")
        s = lax.axis_index("s")
        _zero_acc(zero_v, acc_sh, s, _F)
        plsc.subcore_barrier()
        _edge_pipeline(c, src_hbm, dst_hbm, s * e_sub, nchunks,
                       (hlo_hbm, hhi_hbm), acc_sh, srcv, dstv, rowsv,
                       sems[:_SI], sems[_SI:_SI + _SR], sems[_SI + _SR:])
        plsc.subcore_barrier()

        @pl.when(c == 0)
        def _():
            _writeout(acc_sh, alo_hbm, s)

        @pl.when(c == 1)
        def _():
            _writeout(acc_sh, ahi_hbm, s)

    return k(h_lo, h_hi, src, dst)


def _relu_mm_body(x_ref, w_ref, b_ref, o_ref):
    o_ref[...] = jnp.maximum(
        jnp.dot(x_ref[...], w_ref[...], preferred_element_type=jnp.float32)
        + b_ref[...], 0.0)


def _tc_relu_mm(x, w, b):
    kdim = x.shape[1]
    return pl.pallas_call(
        _relu_mm_body,
        grid=(_NB,),
        in_specs=[
            pl.BlockSpec((_R, kdim), lambda i: (i, 0)),
            pl.BlockSpec((kdim, _H), lambda i: (0, 0)),
            pl.BlockSpec((1, _H), lambda i: (0, 0)),
        ],
        out_specs=pl.BlockSpec((_R, _H), lambda i: (i, 0)),
        out_shape=jax.ShapeDtypeStruct((_N, _H), jnp.float32),
    )(x, w, b)


def _res2_body(hlo_ref, hhi_ref, w_ref, b_ref, o_ref):
    w = w_ref[...]
    acc = (jnp.dot(hlo_ref[...], w[:_F, :], preferred_element_type=jnp.float32)
           + jnp.dot(hhi_ref[...], w[_F:, :], preferred_element_type=jnp.float32))
    o_ref[...] = jnp.maximum(acc + b_ref[...], 0.0)


def _tc_res2(h_lo, h_hi, w, b):
    return pl.pallas_call(
        _res2_body,
        grid=(_NB,),
        in_specs=[
            pl.BlockSpec((_R, _F), lambda i: (i, 0)),
            pl.BlockSpec((_R, _F), lambda i: (i, 0)),
            pl.BlockSpec((_H, _H), lambda i: (0, 0)),
            pl.BlockSpec((1, _H), lambda i: (0, 0)),
        ],
        out_specs=pl.BlockSpec((_R, _H), lambda i: (i, 0)),
        out_shape=jax.ShapeDtypeStruct((_N, _H), jnp.float32),
    )(h_lo, h_hi, w, b)


def _layer1_body(p0_ref, p1_ref, r1_ref, w_ref, b_ref, lo_ref, hi_ref):
    agg = p0_ref[...] + p1_ref[...]
    h = jnp.maximum(
        jnp.dot(agg, w_ref[...], preferred_element_type=jnp.float32)
        + b_ref[...], 0.0) + r1_ref[...]
    lo_ref[...] = h[:, :_F]
    hi_ref[...] = h[:, _F:]


def _tc_layer1(p0, p1, r1, w, b):
    return pl.pallas_call(
        _layer1_body,
        grid=(_NB,),
        in_specs=[
            pl.BlockSpec((_R, _F), lambda i: (i, 0)),
            pl.BlockSpec((_R, _F), lambda i: (i, 0)),
            pl.BlockSpec((_R, _H), lambda i: (i, 0)),
            pl.BlockSpec((_F, _H), lambda i: (0, 0)),
            pl.BlockSpec((1, _H), lambda i: (0, 0)),
        ],
        out_specs=[
            pl.BlockSpec((_R, _F), lambda i: (i, 0)),
            pl.BlockSpec((_R, _F), lambda i: (i, 0)),
        ],
        out_shape=[
            jax.ShapeDtypeStruct((_N, _F), jnp.float32),
            jax.ShapeDtypeStruct((_N, _F), jnp.float32),
        ],
    )(p0, p1, r1, w, b)


def _final_body(alo_ref, ahi_ref, r2_ref, w2_ref, b2_ref, wa_ref, ba_ref,
                idrow_ref, idcol_ref, osum_ref, omax_ref):
    i = pl.program_id(0)
    w2 = w2_ref[...]
    h2 = jnp.maximum(
        jnp.dot(alo_ref[...], w2[:_F, :], preferred_element_type=jnp.float32)
        + jnp.dot(ahi_ref[...], w2[_F:, :], preferred_element_type=jnp.float32)
        + b2_ref[...], 0.0) + r2_ref[...]
    aw = jax.nn.sigmoid(
        jnp.sum(h2 * wa_ref[...], axis=1, keepdims=True) + ba_ref[...])

    ids_row = idrow_ref[0]
    idcol = idcol_ref[...]
    iota = lax.broadcasted_iota(jnp.int32, (_G, _R), 0)
    onehot = (iota == ids_row).astype(jnp.float32)
    wsum = jnp.dot(onehot, h2 * aw, preferred_element_type=jnp.float32,
                precision=lax.Precision.HIGHEST)

    x = h2
    sh = 1
    while sh < _R:
        xs = jnp.concatenate([jnp.zeros((sh, _H), jnp.float32), x[:_R - sh]],
                             axis=0)
        ics = jnp.concatenate(
            [jnp.full((sh, 1), -1, jnp.int32), idcol[:_R - sh]], axis=0)
        x = jnp.where(idcol == ics, jnp.maximum(x, xs), x)
        sh *= 2
    idn = jnp.concatenate([idcol[1:], jnp.full((1, 1), -1, jnp.int32)], axis=0)
    y = jnp.where(idcol != idn, x, 0.0)
    idn_row = jnp.concatenate(
        [ids_row[:, 1:], jnp.full((1, 1), -1, jnp.int32)], axis=1)
    flag_row = (ids_row != idn_row).astype(jnp.float32)
    onehot_f = onehot * flag_row
    cnt = jnp.sum(onehot_f, axis=1, keepdims=True)
    pmax = jnp.dot(onehot_f, y, preferred_element_type=jnp.float32,
                precision=lax.Precision.HIGHEST)
    pmax = jnp.where(cnt > 0.5, pmax, -jnp.inf)

    @pl.when(i == 0)
    def _():
        osum_ref[...] = jnp.zeros((_G, _H), jnp.float32)
        omax_ref[...] = jnp.full((_G, _H), -jnp.inf, jnp.float32)

    osum_ref[...] += wsum
    omax_ref[...] = jnp.maximum(omax_ref[...], pmax)


def _tc_final(a_lo, a_hi, r2, w2, b2, wa, ba, ids_row3, ids_col):
    return pl.pallas_call(
        _final_body,
        grid=(_NB,),
        in_specs=[
            pl.BlockSpec((_R, _F), lambda i: (i, 0)),
            pl.BlockSpec((_R, _F), lambda i: (i, 0)),
            pl.BlockSpec((_R, _H), lambda i: (i, 0)),
            pl.BlockSpec((_H, _H), lambda i: (0, 0)),
            pl.BlockSpec((1, _H), lambda i: (0, 0)),
            pl.BlockSpec((1, _H), lambda i: (0, 0)),
            pl.BlockSpec((1, 1), lambda i: (0, 0)),
            pl.BlockSpec((1, 1, _R), lambda i: (i, 0, 0)),
            pl.BlockSpec((_R, 1), lambda i: (i, 0)),
        ],
        out_specs=[
            pl.BlockSpec((_G, _H), lambda i: (0, 0)),
            pl.BlockSpec((_G, _H), lambda i: (0, 0)),
        ],
        out_shape=[
            jax.ShapeDtypeStruct((_G, _H), jnp.float32),
            jax.ShapeDtypeStruct((_G, _H), jnp.float32),
        ],
    )(a_lo, a_hi, r2, w2, b2, wa, ba, ids_row3, ids_col)


def kernel(feats, edge_index, node_graph_ids, W1, b1, Wres1, bres1,
           W2, b2, Wres2, bres2, w_atom, b_atom):
    src = edge_index[0]
    dst = edge_index[1]
    b1r = b1.reshape(1, _H)
    bres1r = bres1.reshape(1, _H)
    b2r = b2.reshape(1, _H)
    bres2r = bres2.reshape(1, _H)
    wa = w_atom.reshape(1, _H)
    bar = b_atom.reshape(1, 1)
    ids_row3 = node_graph_ids.reshape(_NB, 1, _R)
    ids_col = node_graph_ids.reshape(_N, 1)
    p0, p1 = _sc_agg_feats(feats, src, dst)
    r1 = _tc_relu_mm(feats, Wres1, bres1r)
    h_lo, h_hi = _tc_layer1(p0, p1, r1, W1, b1r)
    a_lo, a_hi = _sc_agg_hidden(h_lo, h_hi, src, dst)
    r2 = _tc_res2(h_lo, h_hi, Wres2, bres2r)
    osum, omax = _tc_final(a_lo, a_hi, r2, W2, b2r, wa, bar, ids_row3, ids_col)
    return jnp.concatenate([osum, omax], axis=1)
